# baseline (speedup 1.0000x reference)
import functools

import jax
import jax.numpy as jnp
from jax import lax
from jax.experimental import pallas as pl
from jax.experimental.pallas import tpu as pltpu
from jax.experimental.pallas import tpu_sc as plsc

LN_EPS = 1e-12

_N_TOK = 8192
_D = 128
_H = 2048
_NW = 32
_N_CHUNKS = 2
_TPC = _N_TOK // _N_CHUNKS
_TPW = _TPC // _NW
_TS = 1024
_BPC = _TPC // _TS


def _sc_gather_chunk(table, ids2d, chunk):
    mesh = plsc.VectorSubcoreMesh(core_axis_name="c", subcore_axis_name="s")
    seq = ids2d.shape[1]
    per_row = seq // _TPW
    n_sub = _TPW // 128

    @functools.partial(
        pl.kernel,
        mesh=mesh,
        out_type=jax.ShapeDtypeStruct((_TPC, _D), jnp.float32),
        scratch_types=[
            pltpu.VMEM((n_sub, 128), jnp.int32),
            pltpu.VMEM((_TPW, _D), jnp.float32),
            pltpu.SemaphoreType.DMA,
        ],
    )
    def k(table_hbm, idx_hbm, out_hbm, idx_v, rows_v, sem):
        wid = lax.axis_index("s") * 2 + lax.axis_index("c")
        tok0 = chunk * _TPC + wid * _TPW
        row = tok0 // seq
        col = tok0 % seq
        for j in range(n_sub):
            pltpu.sync_copy(idx_hbm.at[row, pl.ds(col + j * 128, 128)],
                            idx_v.at[j])
        copies = []
        for j in range(n_sub):
            copies.append(
                pltpu.async_copy(
                    table_hbm.at[idx_v.at[j]],
                    rows_v.at[pl.ds(j * 128, 128)],
                    sem,
                )
            )
        for cp in copies:
            cp.wait()
        pltpu.sync_copy(rows_v, out_hbm.at[pl.ds(wid * _TPW, _TPW)])

    return k(table, ids2d)


def _tc_tail_chunk(g, ttf, type_table, pos_table, ln_scale, ln_bias, W, b,
                   chunk, prev_out):
    pos_blocks = 2048 // _TS
    blk0 = chunk * _BPC

    def body(g_ref, tt_ref, type_ref, pos_ref, sc_ref, bi_ref, w_ref,
             bias_ref, *rest):
        o_ref = rest[-1]
        gv = g_ref[...]
        tt = tt_ref[...]
        t0 = type_ref[0:1, :]
        t1 = type_ref[1:2, :]
        te = t0 + tt * (t1 - t0)
        total = gv + te + pos_ref[...]
        mean = jnp.mean(total, axis=-1, keepdims=True)
        cent = total - mean
        var = jnp.mean(cent * cent, axis=-1, keepdims=True)
        xn = cent * lax.rsqrt(var + LN_EPS)
        xn = xn * sc_ref[...] + bi_ref[...]
        o_ref[...] = (
            jnp.dot(xn, w_ref[...], preferred_element_type=jnp.float32)
            + bias_ref[...]
        )

    in_specs = [
        pl.BlockSpec((_TS, _D), lambda i: (i, 0)),
        pl.BlockSpec((_TS, 1), lambda i: (i + blk0, 0)),
        pl.BlockSpec((2, _D), lambda i: (0, 0)),
        pl.BlockSpec((_TS, _D), lambda i: ((i + blk0) % pos_blocks, 0)),
        pl.BlockSpec((1, _D), lambda i: (0, 0)),
        pl.BlockSpec((1, _D), lambda i: (0, 0)),
        pl.BlockSpec((_D, _H), lambda i: (0, 0)),
        pl.BlockSpec((1, _H), lambda i: (0, 0)),
    ]
    args = [g, ttf, type_table, pos_table, ln_scale, ln_bias, W, b]
    aliases = {}
    if prev_out is not None:
        in_specs.append(pl.BlockSpec(memory_space=pl.ANY))
        args.append(prev_out)
        aliases = {8: 0}
    return pl.pallas_call(
        body,
        grid=(_BPC,),
        in_specs=in_specs,
        out_specs=pl.BlockSpec((_TS, _H), lambda i: (i + blk0, 0)),
        out_shape=jax.ShapeDtypeStruct((_N_TOK, _H), jnp.float32),
        input_output_aliases=aliases,
    )(*args)


def kernel(ids, token_type_ids, emb_table, type_table, pos_table, ln_scale,
           ln_bias, W, b):
    B, S = ids.shape
    ids32 = ids.astype(jnp.int32)
    ttf = token_type_ids.astype(jnp.float32).reshape(_N_TOK, 1)
    ln_scale = ln_scale.reshape(1, _D)
    ln_bias = ln_bias.reshape(1, _D)
    b2 = b.reshape(1, _H)

    gs = [_sc_gather_chunk(emb_table, ids32, c) for c in range(_N_CHUNKS)]
    out = None
    for c in range(_N_CHUNKS):
        out = _tc_tail_chunk(gs[c], ttf, type_table, pos_table, ln_scale,
                             ln_bias, W, b2, c, out)
    return out.reshape(B, S, _H)


# speedup vs baseline: 1.0381x; 1.0381x over previous
import functools

import jax
import jax.numpy as jnp
from jax import lax
from jax.experimental import pallas as pl
from jax.experimental.pallas import tpu as pltpu
from jax.experimental.pallas import tpu_sc as plsc

LN_EPS = 1e-12

_N_TOK = 8192
_D = 128
_H = 2048
_NW = 32
_TPW = _N_TOK // _NW
_TS = 1024
_NB = _N_TOK // _TS


def _sc_gather(table, ids2d):
    mesh = plsc.VectorSubcoreMesh(core_axis_name="c", subcore_axis_name="s")
    seq = ids2d.shape[1]
    n_sub = _TPW // 128

    @functools.partial(
        pl.kernel,
        mesh=mesh,
        out_type=jax.ShapeDtypeStruct((_N_TOK, _D), jnp.float32),
        scratch_types=[
            pltpu.VMEM((n_sub, 128), jnp.int32),
            pltpu.VMEM((_TPW, _D), jnp.float32),
            pltpu.SemaphoreType.DMA,
        ],
    )
    def k(table_hbm, idx_hbm, out_hbm, idx_v, rows_v, sem):
        wid = lax.axis_index("s") * 2 + lax.axis_index("c")
        tok0 = wid * _TPW
        row = tok0 // seq
        col = tok0 % seq
        for j in range(n_sub):
            pltpu.sync_copy(idx_hbm.at[row, pl.ds(col + j * 128, 128)],
                            idx_v.at[j])
        copies = []
        for j in range(n_sub):
            copies.append(
                pltpu.async_copy(
                    table_hbm.at[idx_v.at[j]],
                    rows_v.at[pl.ds(j * 128, 128)],
                    sem,
                )
            )
        for cp in copies:
            cp.wait()
        pltpu.sync_copy(rows_v, out_hbm.at[pl.ds(tok0, _TPW)])

    return k(table, ids2d)


def _tc_tail(g, ttf, type_table, pos_table, ln_scale, ln_bias, W, b):
    pos_blocks = 2048 // _TS

    def body(g_ref, tt_ref, type_ref, pos_ref, sc_ref, bi_ref, w_ref,
             bias_ref, o_hbm, buf, sems):
        i = pl.program_id(0)
        par = i % 2

        def drain(p, blk):
            pltpu.make_async_copy(
                buf.at[p], o_hbm.at[pl.ds(blk * _TS, _TS), :], sems.at[p]
            ).wait()

        @pl.when(i >= 2)
        def _():
            drain(par, i - 2)

        gv = g_ref[...]
        tt = tt_ref[...]
        t0 = type_ref[0:1, :]
        t1 = type_ref[1:2, :]
        te = t0 + tt * (t1 - t0)
        total = gv + te + pos_ref[...]
        mean = jnp.mean(total, axis=-1, keepdims=True)
        cent = total - mean
        var = jnp.mean(cent * cent, axis=-1, keepdims=True)
        xn = cent * lax.rsqrt(var + LN_EPS)
        xn = xn * sc_ref[...] + bi_ref[...]
        res = (
            jnp.dot(xn, w_ref[...], preferred_element_type=jnp.float32)
            + bias_ref[...]
        )

        for p in range(2):
            @pl.when(par == p)
            def _(p=p):
                buf[p, :, :] = res
                pltpu.make_async_copy(
                    buf.at[p], o_hbm.at[pl.ds(i * _TS, _TS), :], sems.at[p]
                ).start()

        @pl.when(i == _NB - 1)
        def _():
            drain(1 - par, i - 1)
            drain(par, i)

    return pl.pallas_call(
        body,
        grid=(_NB,),
        in_specs=[
            pl.BlockSpec((_TS, _D), lambda i: (i, 0)),
            pl.BlockSpec((_TS, 1), lambda i: (i, 0)),
            pl.BlockSpec((2, _D), lambda i: (0, 0)),
            pl.BlockSpec((_TS, _D), lambda i: (i % pos_blocks, 0)),
            pl.BlockSpec((1, _D), lambda i: (0, 0)),
            pl.BlockSpec((1, _D), lambda i: (0, 0)),
            pl.BlockSpec((_D, _H), lambda i: (0, 0)),
            pl.BlockSpec((1, _H), lambda i: (0, 0)),
        ],
        out_specs=pl.BlockSpec(memory_space=pl.ANY),
        out_shape=jax.ShapeDtypeStruct((_N_TOK, _H), jnp.float32),
        scratch_shapes=[
            pltpu.VMEM((2, _TS, _H), jnp.float32),
            pltpu.SemaphoreType.DMA((2,)),
        ],
    )(g, ttf, type_table, pos_table, ln_scale, ln_bias, W, b)


def kernel(ids, token_type_ids, emb_table, type_table, pos_table, ln_scale,
           ln_bias, W, b):
    B, S = ids.shape
    ids32 = ids.astype(jnp.int32)
    ttf = token_type_ids.astype(jnp.float32).reshape(_N_TOK, 1)
    g = _sc_gather(emb_table, ids32)
    hidden = _tc_tail(
        g, ttf, type_table, pos_table,
        ln_scale.reshape(1, _D), ln_bias.reshape(1, _D),
        W, b.reshape(1, _H),
    )
    return hidden.reshape(B, S, _H)


# speedup vs baseline: 1.0800x; 1.0404x over previous
import functools

import jax
import jax.numpy as jnp
from jax import lax
from jax.experimental import pallas as pl
from jax.experimental.pallas import tpu as pltpu
from jax.experimental.pallas import tpu_sc as plsc

LN_EPS = 1e-12

_N_TOK = 8192
_D = 128
_H = 2048
_NW = 32
_TPW = _N_TOK // _NW
_TS = 1024
_NB = _N_TOK // _TS


def _sc_gather(table, ids2d):
    mesh = plsc.VectorSubcoreMesh(core_axis_name="c", subcore_axis_name="s")
    seq = ids2d.shape[1]
    n_sub = _TPW // 128

    @functools.partial(
        pl.kernel,
        mesh=mesh,
        out_type=jax.ShapeDtypeStruct((_N_TOK, _D), jnp.float32),
        scratch_types=[
            pltpu.VMEM((n_sub, 128), jnp.int32),
            pltpu.VMEM((_TPW, _D), jnp.float32),
            pltpu.SemaphoreType.DMA,
            pltpu.SemaphoreType.DMA,
            pltpu.SemaphoreType.DMA,
        ],
    )
    def k(table_hbm, idx_hbm, out_hbm, idx_v, rows_v, sem_i, sem_g, sem_s):
        wid = lax.axis_index("s") * 2 + lax.axis_index("c")
        tok0 = wid * _TPW
        row = tok0 // seq
        col = tok0 % seq
        idx_cp = [
            pltpu.async_copy(idx_hbm.at[row, pl.ds(col + j * 128, 128)],
                             idx_v.at[j], sem_i)
            for j in range(n_sub)
        ]
        gather_cp = []
        for j in range(n_sub):
            idx_cp[j].wait()
            gather_cp.append(
                pltpu.async_copy(
                    table_hbm.at[idx_v.at[j]],
                    rows_v.at[pl.ds(j * 128, 128)],
                    sem_g,
                )
            )
        scatter_cp = []
        for j in range(n_sub):
            gather_cp[j].wait()
            scatter_cp.append(
                pltpu.async_copy(
                    rows_v.at[pl.ds(j * 128, 128)],
                    out_hbm.at[pl.ds(tok0 + j * 128, 128)],
                    sem_s,
                )
            )
        for cp in scatter_cp:
            cp.wait()

    return k(table, ids2d)


def _tc_tail(g, ttf, type_table, pos_table, ln_scale, ln_bias, W, b):
    pos_blocks = 2048 // _TS

    def body(g_ref, tt_ref, type_ref, pos_ref, sc_ref, bi_ref, w_ref,
             bias_ref, o_ref):
        gv = g_ref[...]
        tt = tt_ref[...]
        t0 = type_ref[0:1, :]
        t1 = type_ref[1:2, :]
        te = t0 + tt * (t1 - t0)
        total = gv + te + pos_ref[...]
        mean = jnp.mean(total, axis=-1, keepdims=True)
        cent = total - mean
        var = jnp.mean(cent * cent, axis=-1, keepdims=True)
        xn = cent * lax.rsqrt(var + LN_EPS)
        xn = xn * sc_ref[...] + bi_ref[...]
        o_ref[...] = (
            jnp.dot(xn, w_ref[...], preferred_element_type=jnp.float32)
            + bias_ref[...]
        )

    return pl.pallas_call(
        body,
        grid=(_NB,),
        in_specs=[
            pl.BlockSpec((_TS, _D), lambda i: (i, 0)),
            pl.BlockSpec((_TS, 1), lambda i: (i, 0)),
            pl.BlockSpec((2, _D), lambda i: (0, 0)),
            pl.BlockSpec((_TS, _D), lambda i: (i % pos_blocks, 0)),
            pl.BlockSpec((1, _D), lambda i: (0, 0)),
            pl.BlockSpec((1, _D), lambda i: (0, 0)),
            pl.BlockSpec((_D, _H), lambda i: (0, 0)),
            pl.BlockSpec((1, _H), lambda i: (0, 0)),
        ],
        out_specs=pl.BlockSpec((_TS, _H), lambda i: (i, 0)),
        out_shape=jax.ShapeDtypeStruct((_N_TOK, _H), jnp.float32),
    )(g, ttf, type_table, pos_table, ln_scale, ln_bias, W, b)


def kernel(ids, token_type_ids, emb_table, type_table, pos_table, ln_scale,
           ln_bias, W, b):
    B, S = ids.shape
    ids32 = ids.astype(jnp.int32)
    ttf = token_type_ids.astype(jnp.float32).reshape(_N_TOK, 1)
    g = _sc_gather(emb_table, ids32)
    hidden = _tc_tail(
        g, ttf, type_table, pos_table,
        ln_scale.reshape(1, _D), ln_bias.reshape(1, _D),
        W, b.reshape(1, _H),
    )
    return hidden.reshape(B, S, _H)
